# baseline (speedup 1.0000x reference)
import jax
import jax.numpy as jnp
import numpy as np
from jax.experimental import pallas as pl
from jax.experimental.pallas import tpu as pltpu

DIM = 128
NUM_ACTION = 100000
BATCH = 1024
CB = 1024
NBLK = (NUM_ACTION + CB - 1) // CB

_TINY = np.float32(np.finfo(np.float32).tiny)
_KS1 = np.uint32(42)
_KS2 = np.uint32(0x1BD11BDA) ^ np.uint32(42)
_ROT_A = (13, 15, 26, 6)
_ROT_B = (17, 29, 16, 24)


def _rotl(x, d):
    return (x << np.uint32(d)) | (x >> np.uint32(32 - d))


def _rounds(x0, x1, rots):
    for r in rots:
        x0 = x0 + x1
        x1 = _rotl(x1, r)
        x1 = x0 ^ x1
    return x0, x1


def _gumbel(flat_idx_u32):
    x0 = jnp.zeros_like(flat_idx_u32)
    x1 = flat_idx_u32 + _KS1
    x0, x1 = _rounds(x0, x1, _ROT_A)
    x0 = x0 + _KS1
    x1 = x1 + (_KS2 + np.uint32(1))
    x0, x1 = _rounds(x0, x1, _ROT_B)
    x0 = x0 + _KS2
    x1 = x1 + np.uint32(2)
    x0, x1 = _rounds(x0, x1, _ROT_A)
    x1 = x1 + (_KS1 + np.uint32(3))
    x0, x1 = _rounds(x0, x1, _ROT_B)
    x0 = x0 + _KS1
    x1 = x1 + (_KS2 + np.uint32(4))
    x0, x1 = _rounds(x0, x1, _ROT_A)
    x0 = x0 + _KS2
    x1 = x1 + np.uint32(5)
    bits = x0 ^ x1
    float_bits = (bits >> np.uint32(9)) | np.uint32(0x3F800000)
    floats = jax.lax.bitcast_convert_type(float_bits, jnp.float32) - np.float32(1.0)
    u = jnp.maximum(_TINY, floats * (np.float32(1.0) - _TINY) + _TINY)
    return -jnp.log(-jnp.log(u))


def _body(feature_ref, w1t_ref, b1_ref, w2t_ref, b2_ref,
          logits_ref, draw_ref, h_sc, bestv_sc, besti_sc):
    j = pl.program_id(0)

    @pl.when(j == 0)
    def _init():
        h = jnp.dot(feature_ref[...], w1t_ref[...],
                    preferred_element_type=jnp.float32)
        h_sc[...] = jnp.maximum(h + b1_ref[...], 0.0)
        bestv_sc[...] = jnp.full((BATCH, 1), -jnp.inf, jnp.float32)
        besti_sc[...] = jnp.zeros((BATCH, 1), jnp.int32)

    h = h_sc[...]
    logits = jnp.dot(h, w2t_ref[...], preferred_element_type=jnp.float32)
    logits = logits + b2_ref[...]
    logits_ref[...] = logits

    col = j * CB + jax.lax.broadcasted_iota(jnp.int32, (BATCH, CB), 1)
    row = jax.lax.broadcasted_iota(jnp.int32, (BATCH, CB), 0)
    flat = (row * NUM_ACTION + col).astype(jnp.uint32)
    g = _gumbel(flat)
    v = jnp.where(col < NUM_ACTION, logits + g, -jnp.inf)

    m = jnp.max(v, axis=1, keepdims=True)
    idx = jnp.min(jnp.where(v == m, col, np.int32(2**31 - 1)),
                  axis=1, keepdims=True)
    better = m > bestv_sc[...]
    bestv_sc[...] = jnp.where(better, m, bestv_sc[...])
    besti_sc[...] = jnp.where(better, idx, besti_sc[...])

    @pl.when(j == NBLK - 1)
    def _fin():
        draw_ref[...] = besti_sc[...]


@jax.jit
def kernel(feature, W1, b1, W2, b2):
    w1t = W1.T
    b1r = b1.reshape(1, 8)
    w2t = W2.T
    b2r = b2.reshape(1, NUM_ACTION)
    logits, draw = pl.pallas_call(
        _body,
        grid=(NBLK,),
        in_specs=[
            pl.BlockSpec((BATCH, DIM), lambda j: (0, 0)),
            pl.BlockSpec((DIM, 8), lambda j: (0, 0)),
            pl.BlockSpec((1, 8), lambda j: (0, 0)),
            pl.BlockSpec((8, CB), lambda j: (0, j)),
            pl.BlockSpec((1, CB), lambda j: (0, j)),
        ],
        out_specs=[
            pl.BlockSpec((BATCH, CB), lambda j: (0, j)),
            pl.BlockSpec((BATCH, 1), lambda j: (0, 0)),
        ],
        out_shape=[
            jax.ShapeDtypeStruct((BATCH, NUM_ACTION), jnp.float32),
            jax.ShapeDtypeStruct((BATCH, 1), jnp.int32),
        ],
        scratch_shapes=[
            pltpu.VMEM((BATCH, 8), jnp.float32),
            pltpu.VMEM((BATCH, 1), jnp.float32),
            pltpu.VMEM((BATCH, 1), jnp.int32),
        ],
    )(feature, w1t, b1r, w2t, b2r)
    return (logits, draw)
